# baseline (speedup 1.0000x reference)
import functools

import jax
import jax.numpy as jnp
from jax.experimental import pallas as pl

DIM = 4096
INTER = 14336
BLK = 512


def _ffn_kernel(x_ref, w1_ref, w2_ref, o_ref):
    @pl.when(pl.program_id(0) == 0)
    def _init():
        o_ref[...] = jnp.zeros_like(o_ref)

    h = jax.lax.dot_general(
        x_ref[...], w1_ref[...],
        dimension_numbers=(((1,), (1,)), ((), ())),
        preferred_element_type=jnp.float32,
    )
    h = jnp.maximum(h, 0.0)
    o_ref[...] += jax.lax.dot_general(
        h, w2_ref[...],
        dimension_numbers=(((1,), (1,)), ((), ())),
        preferred_element_type=jnp.float32,
    )


@jax.jit
def kernel(x, W1, W2):
    b, t, d = x.shape
    xt = x.reshape(b * t, d)
    out = pl.pallas_call(
        _ffn_kernel,
        grid=(INTER // BLK,),
        in_specs=[
            pl.BlockSpec((b * t, DIM), lambda i: (0, 0)),
            pl.BlockSpec((BLK, DIM), lambda i: (i, 0)),
            pl.BlockSpec((DIM, BLK), lambda i: (0, i)),
        ],
        out_specs=pl.BlockSpec((b * t, DIM), lambda i: (0, 0)),
        out_shape=jax.ShapeDtypeStruct((b * t, DIM), jnp.float32),
    )(xt, W1, W2)
    return out.reshape(b, t, d)
